# Initial kernel scaffold; baseline (speedup 1.0000x reference)
import jax
import jax.numpy as jnp
from jax.experimental import pallas as pl


def kernel(x, edge_index, edge_weight, W1, b1, W2, b2, cheb_coef):
    raise NotImplementedError("write your pallas kernel here")


# speedup vs baseline: 6.0044x; 6.0044x over previous
import functools

import numpy as np
import jax
import jax.numpy as jnp
from jax import lax
from jax.experimental import pallas as pl
from jax.experimental.pallas import tpu as pltpu
from jax.experimental.pallas import tpu_sc as plsc

N = 10000
E = 320000
D_IN = 128
D_HID = 128
D_OUT = 64
K = 10

NPAD = 10240
NC, NS = 2, 16
NW = NC * NS
EPT = E // NW
CH = 80
NCH = EPT // CH
RPT = NPAD // NS
DPT = NPAD // NS


def _jackson_damp():
    k = np.arange(K + 1, dtype=np.float64)
    c = np.pi / (K + 2)
    damp = ((K + 2 - k) * np.sin(c) * np.cos(k * c)
            + np.cos(c) * np.sin(k * c)) / ((K + 2) * np.sin(c))
    return damp.astype(np.float32)


_DAMP = _jackson_damp()


def _mlp_body(x_ref, w1t_ref, b1_ref, w2t_ref, b2_ref, h_ref):
    h1 = jnp.dot(x_ref[...], w1t_ref[...], preferred_element_type=jnp.float32)
    h1 = h1 + b1_ref[...][None, :]
    h1 = jnp.where(h1 > 0, h1, 0.01 * h1)
    h2 = jnp.dot(h1, w2t_ref[...], preferred_element_type=jnp.float32)
    h_ref[...] = h2 + b2_ref[...][None, :]


def _mlp(x, w1t, b1, w2t, b2):
    R = 1024
    return pl.pallas_call(
        _mlp_body,
        grid=(NPAD // R,),
        in_specs=[
            pl.BlockSpec((R, D_IN), lambda i: (i, 0)),
            pl.BlockSpec((D_IN, D_HID), lambda i: (0, 0)),
            pl.BlockSpec((D_HID,), lambda i: (0,)),
            pl.BlockSpec((D_HID, D_OUT), lambda i: (0, 0)),
            pl.BlockSpec((D_OUT,), lambda i: (0,)),
        ],
        out_specs=pl.BlockSpec((R, D_OUT), lambda i: (i, 0)),
        out_shape=jax.ShapeDtypeStruct((NPAD, D_OUT), jnp.float32),
    )(x, w1t, b1, w2t, b2)


def _deg_body(col_hbm, w_hbm, z_hbm, degp_hbm, colv, wv, deg_sh):
    c = lax.axis_index("c")
    s = lax.axis_index("s")
    wid = c * NS + s
    pltpu.sync_copy(z_hbm.at[pl.ds(s * DPT, DPT)], deg_sh.at[pl.ds(s * DPT, DPT)])
    plsc.subcore_barrier()

    def chunk(j, carry):
        base = wid * EPT + j * CH
        pltpu.sync_copy(col_hbm.at[pl.ds(base, CH)], colv.at[0])
        pltpu.sync_copy(w_hbm.at[pl.ds(base, CH)], wv)
        pltpu.sync_copy(wv, deg_sh.at[colv.at[0]], add=True)
        return carry

    lax.fori_loop(0, NCH, chunk, 0)
    plsc.subcore_barrier()
    pltpu.sync_copy(deg_sh.at[pl.ds(s * DPT, DPT)],
                    degp_hbm.at[c, pl.ds(s * DPT, DPT)])


def _sc_params():
    return pltpu.CompilerParams(needs_layout_passes=False, use_tc_tiling_on_sc=False)


def _deg(col, w, zpad):
    mesh = plsc.VectorSubcoreMesh(core_axis_name="c", subcore_axis_name="s")
    f = pl.kernel(
        _deg_body,
        out_type=jax.ShapeDtypeStruct((NC, NPAD), jnp.float32),
        mesh=mesh,
        compiler_params=_sc_params(),
        scratch_types=[
            pltpu.VMEM((1, CH), jnp.int32),
            pltpu.VMEM((CH,), jnp.float32),
            pltpu.VMEM_SHARED((NPAD,), jnp.float32),
        ],
    )
    return f(col, w, zpad)


def _hop_body(sv_hbm, row_hbm, col_hbm, w_hbm, z_hbm, acc_hbm,
              rowv, colv, wv, rows_v, acc_sh, sv_sh):
    c = lax.axis_index("c")
    s = lax.axis_index("s")
    wid = c * NS + s
    pltpu.sync_copy(z_hbm.at[pl.ds(s * RPT, RPT)], acc_sh.at[pl.ds(s * RPT, RPT)])
    pltpu.sync_copy(sv_hbm.at[pl.ds(s * RPT, RPT)], sv_sh.at[pl.ds(s * RPT, RPT)])
    plsc.subcore_barrier()

    def chunk(j, carry):
        base = wid * EPT + j * CH
        pltpu.sync_copy(row_hbm.at[pl.ds(base, CH)], rowv)
        pltpu.sync_copy(col_hbm.at[pl.ds(base, CH)], colv.at[0])
        pltpu.sync_copy(w_hbm.at[pl.ds(base, CH)], wv)
        pltpu.sync_copy(sv_sh.at[rowv], rows_v)
        for g in range(CH // 16):
            bw16 = wv[pl.ds(g * 16, 16)]
            for i in range(16):
                e = g * 16 + i
                bw = bw16[i]
                for q in range(4):
                    rows_v[e, pl.ds(q * 16, 16)] = rows_v[e, pl.ds(q * 16, 16)] * bw
        pltpu.sync_copy(rows_v, acc_sh.at[colv.at[0]], add=True)
        return carry

    lax.fori_loop(0, NCH, chunk, 0)
    plsc.subcore_barrier()
    pltpu.sync_copy(acc_sh.at[pl.ds(s * RPT, RPT)],
                    acc_hbm.at[c, pl.ds(s * RPT, RPT)])


def _hop(sv, row, col, w, z2):
    mesh = plsc.VectorSubcoreMesh(core_axis_name="c", subcore_axis_name="s")
    f = pl.kernel(
        _hop_body,
        out_type=jax.ShapeDtypeStruct((NC, NPAD, D_OUT), jnp.float32),
        mesh=mesh,
        compiler_params=_sc_params(),
        scratch_types=[
            pltpu.VMEM((CH,), jnp.int32),
            pltpu.VMEM((1, CH), jnp.int32),
            pltpu.VMEM((CH,), jnp.float32),
            pltpu.VMEM((CH, D_OUT), jnp.float32),
            pltpu.VMEM_SHARED((NPAD, D_OUT), jnp.float32),
            pltpu.VMEM_SHARED((NPAD, D_OUT), jnp.float32),
        ],
    )
    return f(sv, row, col, w, z2)


def _prep_body(degp_ref, h_ref, dinv_ref, sv_ref):
    deg = degp_ref[0, :] + degp_ref[1, :]
    dinv = jnp.where(deg > 0, lax.rsqrt(jnp.maximum(deg, 1e-12)), 0.0)
    dinv_ref[...] = dinv[:, None]
    sv_ref[...] = h_ref[...] * dinv[:, None]


def _prep(degp, h):
    R = 1024
    return pl.pallas_call(
        _prep_body,
        grid=(NPAD // R,),
        in_specs=[
            pl.BlockSpec((NC, R), lambda i: (0, i)),
            pl.BlockSpec((R, D_OUT), lambda i: (i, 0)),
        ],
        out_specs=[
            pl.BlockSpec((R, 1), lambda i: (i, 0)),
            pl.BlockSpec((R, D_OUT), lambda i: (i, 0)),
        ],
        out_shape=[
            jax.ShapeDtypeStruct((NPAD, 1), jnp.float32),
            jax.ShapeDtypeStruct((NPAD, D_OUT), jnp.float32),
        ],
    )(degp, h)


def _comb1_body(acc_ref, dinv_ref, h_ref, c01_ref, tx1_ref, sv_ref, out_ref):
    p = (acc_ref[0] + acc_ref[1]) * dinv_ref[...]
    tx1_ref[...] = p
    sv_ref[...] = p * dinv_ref[...]
    out_ref[...] = c01_ref[0, 0] * h_ref[...] + c01_ref[0, 1] * p


def _comb1(acc, dinv, h, c01):
    R = 1024
    return pl.pallas_call(
        _comb1_body,
        grid=(NPAD // R,),
        in_specs=[
            pl.BlockSpec((NC, R, D_OUT), lambda i: (0, i, 0)),
            pl.BlockSpec((R, 1), lambda i: (i, 0)),
            pl.BlockSpec((R, D_OUT), lambda i: (i, 0)),
            pl.BlockSpec(memory_space=pltpu.SMEM),
        ],
        out_specs=[
            pl.BlockSpec((R, D_OUT), lambda i: (i, 0)),
            pl.BlockSpec((R, D_OUT), lambda i: (i, 0)),
            pl.BlockSpec((R, D_OUT), lambda i: (i, 0)),
        ],
        out_shape=[
            jax.ShapeDtypeStruct((NPAD, D_OUT), jnp.float32),
            jax.ShapeDtypeStruct((NPAD, D_OUT), jnp.float32),
            jax.ShapeDtypeStruct((NPAD, D_OUT), jnp.float32),
        ],
    )(acc, dinv, h, c01)


def _comb2_body(acc_ref, dinv_ref, tx0_ref, outp_ref, ck_ref,
                tx2_ref, sv_ref, out_ref):
    p = (acc_ref[0] + acc_ref[1]) * dinv_ref[...]
    t2 = 2.0 * p - tx0_ref[...]
    tx2_ref[...] = t2
    sv_ref[...] = t2 * dinv_ref[...]
    out_ref[...] = outp_ref[...] + ck_ref[0, 0] * t2


def _comb2(acc, dinv, tx0, outp, ck):
    R = 1024
    return pl.pallas_call(
        _comb2_body,
        grid=(NPAD // R,),
        in_specs=[
            pl.BlockSpec((NC, R, D_OUT), lambda i: (0, i, 0)),
            pl.BlockSpec((R, 1), lambda i: (i, 0)),
            pl.BlockSpec((R, D_OUT), lambda i: (i, 0)),
            pl.BlockSpec((R, D_OUT), lambda i: (i, 0)),
            pl.BlockSpec(memory_space=pltpu.SMEM),
        ],
        out_specs=[
            pl.BlockSpec((R, D_OUT), lambda i: (i, 0)),
            pl.BlockSpec((R, D_OUT), lambda i: (i, 0)),
            pl.BlockSpec((R, D_OUT), lambda i: (i, 0)),
        ],
        out_shape=[
            jax.ShapeDtypeStruct((NPAD, D_OUT), jnp.float32),
            jax.ShapeDtypeStruct((NPAD, D_OUT), jnp.float32),
            jax.ShapeDtypeStruct((NPAD, D_OUT), jnp.float32),
        ],
    )(acc, dinv, tx0, outp, ck)


def kernel(x, edge_index, edge_weight, W1, b1, W2, b2, cheb_coef):
    row = edge_index[0]
    col = edge_index[1]
    h = _mlp(x, W1.T, b1, W2.T, b2)

    zpad = jnp.zeros((NPAD,), jnp.float32)
    z2 = jnp.zeros((NPAD, D_OUT), jnp.float32)
    degp = _deg(col, edge_weight, zpad)
    dinv, sv = _prep(degp, h)

    coefs = cheb_coef * jnp.asarray(_DAMP)

    acc = _hop(sv, row, col, edge_weight, z2)
    tx1, sv, out = _comb1(acc, dinv, h, coefs[0:2].reshape(1, 2))
    tx0 = h
    for k in range(2, K + 1):
        acc = _hop(sv, row, col, edge_weight, z2)
        tx2, sv, out = _comb2(acc, dinv, tx0, out, coefs[k].reshape(1, 1))
        tx0, tx1 = tx1, tx2
    return out[:N]
